# baseline (speedup 1.0000x reference)
import functools

import jax
import jax.numpy as jnp
from jax import lax
from jax.experimental import pallas as pl
from jax.experimental.pallas import tpu as pltpu
from jax.experimental.pallas import tpu_sc as plsc

_NUM_FRAMES = 100000
_NUM_CHANNELS = 64
_BATCH = 16384

_info = plsc.get_sparse_core_info()
_NC, _NS, _L = _info.num_cores, _info.num_subcores, _info.num_lanes
_NW = _NC * _NS
_CPW = _NUM_CHANNELS // _NW
_BCHUNK = 4096
_NBCHUNK = _BATCH // _BCHUNK


@functools.partial(
    pl.kernel,
    mesh=plsc.VectorSubcoreMesh(core_axis_name="c", subcore_axis_name="s"),
    out_type=jax.ShapeDtypeStruct((_NUM_CHANNELS, _BATCH), jnp.float32),
    scratch_types=[
        pltpu.VMEM((_NUM_FRAMES,), jnp.float32),
        pltpu.VMEM((2, _BCHUNK), jnp.int32),
        pltpu.VMEM((2, _BCHUNK), jnp.float32),
        pltpu.SemaphoreType.DMA,
        pltpu.SemaphoreType.DMA,
        pltpu.SemaphoreType.DMA,
    ],
    compiler_params=pltpu.CompilerParams(
        disable_bounds_checks=True,
        disable_semaphore_checks=True,
        needs_layout_passes=False,
    ),
)
def _gather_kernel(ids_hbm, data_t_hbm, out_t_hbm, row_v, idx2_v, val2_v,
                   rsem, isem, osem):
    wid = lax.axis_index("s") * _NC + lax.axis_index("c")
    pending_writes = []

    for cc in range(_CPW):
        ch = wid * _CPW + cc
        row_cp = pltpu.async_copy(data_t_hbm.at[ch], row_v, rsem)
        ids_cps = [
            pltpu.async_copy(
                ids_hbm.at[pl.ds(b * _BCHUNK, _BCHUNK)], idx2_v.at[b % 2], isem
            )
            for b in range(2)
        ]
        row_cp.wait()
        for b in range(_NBCHUNK):
            cur = b % 2
            ids_cps[b].wait()
            if len(pending_writes) >= 2:
                pending_writes.pop(0).wait()

            def gather_block(k):
                idx = idx2_v[cur, pl.ds(k * _L, _L)]
                val2_v[cur, pl.ds(k * _L, _L)] = plsc.load_gather(row_v, [idx])

            pl.loop(0, _BCHUNK // _L, unroll=8)(gather_block)
            if b + 2 < _NBCHUNK:
                ids_cps.append(
                    pltpu.async_copy(
                        ids_hbm.at[pl.ds((b + 2) * _BCHUNK, _BCHUNK)],
                        idx2_v.at[cur],
                        isem,
                    )
                )
            pending_writes.append(
                pltpu.async_copy(
                    val2_v.at[cur],
                    out_t_hbm.at[ch, pl.ds(b * _BCHUNK, _BCHUNK)],
                    osem,
                )
            )
    for w in pending_writes:
        w.wait()


def kernel(ids, data):
    out_t = _gather_kernel(ids, data.T)
    return out_t.T


# speedup vs baseline: 1.0004x; 1.0004x over previous
import functools

import jax
import jax.numpy as jnp
from jax import lax
from jax.experimental import pallas as pl
from jax.experimental.pallas import tpu as pltpu
from jax.experimental.pallas import tpu_sc as plsc

_NUM_FRAMES = 100000
_NUM_CHANNELS = 64
_BATCH = 16384

_info = plsc.get_sparse_core_info()
_NC, _NS, _L = _info.num_cores, _info.num_subcores, _info.num_lanes
_NW = _NC * _NS
_CPW = _NUM_CHANNELS // _NW
_BCHUNK = 4096
_NBCHUNK = _BATCH // _BCHUNK


@functools.partial(
    pl.kernel,
    mesh=plsc.VectorSubcoreMesh(core_axis_name="c", subcore_axis_name="s"),
    out_type=jax.ShapeDtypeStruct((_NUM_CHANNELS, _BATCH), jnp.float32),
    scratch_types=[
        pltpu.VMEM((_NUM_FRAMES,), jnp.float32),
        pltpu.VMEM((2, _BCHUNK), jnp.int32),
        pltpu.VMEM((2, _BCHUNK), jnp.float32),
        pltpu.SemaphoreType.DMA,
        pltpu.SemaphoreType.DMA,
        pltpu.SemaphoreType.DMA,
    ],
    compiler_params=pltpu.CompilerParams(
        disable_bounds_checks=True,
        disable_semaphore_checks=True,
        needs_layout_passes=False,
    ),
)
def _gather_kernel(ids_hbm, data_t_hbm, out_t_hbm, row_v, idx2_v, val2_v,
                   rsem, isem, osem):
    wid = lax.axis_index("s") * _NC + lax.axis_index("c")
    pending_writes = []

    for cc in range(_CPW):
        ch = wid * _CPW + cc
        row_cp = pltpu.async_copy(data_t_hbm.at[ch], row_v, rsem)
        ids_cps = [
            pltpu.async_copy(
                ids_hbm.at[pl.ds(b * _BCHUNK, _BCHUNK)], idx2_v.at[b % 2], isem
            )
            for b in range(2)
        ]
        with jax.named_scope("row_stream_wait"):
            row_cp.wait()
        for b in range(_NBCHUNK):
            cur = b % 2
            ids_cps[b].wait()
            if len(pending_writes) >= 2:
                pending_writes.pop(0).wait()

            def gather_block(k):
                idx = idx2_v[cur, pl.ds(k * _L, _L)]
                val2_v[cur, pl.ds(k * _L, _L)] = plsc.load_gather(row_v, [idx])

            with jax.named_scope("gather_loop"):
                pl.loop(0, _BCHUNK // _L, unroll=8)(gather_block)
            if b + 2 < _NBCHUNK:
                ids_cps.append(
                    pltpu.async_copy(
                        ids_hbm.at[pl.ds((b + 2) * _BCHUNK, _BCHUNK)],
                        idx2_v.at[cur],
                        isem,
                    )
                )
            pending_writes.append(
                pltpu.async_copy(
                    val2_v.at[cur],
                    out_t_hbm.at[ch, pl.ds(b * _BCHUNK, _BCHUNK)],
                    osem,
                )
            )
    for w in pending_writes:
        w.wait()


def kernel(ids, data):
    out_t = _gather_kernel(ids, data.T)
    return out_t.T


# speedup vs baseline: 1.4625x; 1.4620x over previous
import functools

import jax
import jax.numpy as jnp
from jax import lax
from jax.experimental import pallas as pl
from jax.experimental.pallas import tpu as pltpu
from jax.experimental.pallas import tpu_sc as plsc

_NUM_FRAMES = 100000
_NUM_CHANNELS = 64
_BATCH = 16384

_info = plsc.get_sparse_core_info()
_NC, _NS, _L = _info.num_cores, _info.num_subcores, _info.num_lanes
_NW = _NC * _NS
_CPW = _NUM_CHANNELS // _NW
_BCHUNK = 4096
_NBCHUNK = _BATCH // _BCHUNK


@functools.partial(
    pl.kernel,
    mesh=plsc.VectorSubcoreMesh(core_axis_name="c", subcore_axis_name="s"),
    out_type=jax.ShapeDtypeStruct((_NUM_CHANNELS, _BATCH), jnp.float32),
    scratch_types=[
        pltpu.VMEM((_NUM_FRAMES,), jnp.float32),
        pltpu.VMEM((_BATCH,), jnp.int32),
        pltpu.VMEM((2, _BCHUNK), jnp.float32),
        pltpu.SemaphoreType.DMA,
        pltpu.SemaphoreType.DMA,
        pltpu.SemaphoreType.DMA,
    ],
    compiler_params=pltpu.CompilerParams(
        disable_bounds_checks=True,
        disable_semaphore_checks=True,
        needs_layout_passes=False,
    ),
)
def _gather_kernel(ids_hbm, data_t_hbm, out_t_hbm, row_v, idx_v, val2_v,
                   rsem, isem, osem):
    wid = lax.axis_index("s") * _NC + lax.axis_index("c")
    ids_cp = pltpu.async_copy(ids_hbm, idx_v, isem)
    ids_waited = False
    pending_writes = []

    for cc in range(_CPW):
        ch = wid * _CPW + cc
        pltpu.sync_copy(data_t_hbm.at[ch], row_v)
        if not ids_waited:
            ids_cp.wait()
            ids_waited = True
        for b in range(_NBCHUNK):
            cur = b % 2
            if len(pending_writes) >= 2:
                pending_writes.pop(0).wait()

            def gather_block(k):
                idx = idx_v[pl.ds(b * _BCHUNK + k * _L, _L)]
                val2_v[cur, pl.ds(k * _L, _L)] = plsc.load_gather(row_v, [idx])

            plsc.parallel_loop(0, _BCHUNK // _L, unroll=8)(gather_block)
            pending_writes.append(
                pltpu.async_copy(
                    val2_v.at[cur],
                    out_t_hbm.at[ch, pl.ds(b * _BCHUNK, _BCHUNK)],
                    osem,
                )
            )
    for w in pending_writes:
        w.wait()


def kernel(ids, data):
    out_t = _gather_kernel(ids, data.T)
    return out_t.T
